# baseline (speedup 1.0000x reference)
import functools

import jax
import jax.numpy as jnp
from jax.experimental import pallas as pl

N = 10000
VAR = 4
NIN = 256
NZ = 100
NC = 10

_BM_PREP = 1000
_BM_G = 200


def _mm(a, b):
    return jax.lax.dot_general(a, b, (((1,), (0,)), ((), ())),
                               preferred_element_type=jnp.float32)


def _conv0(x, t0_ref, c0b_ref):
    pc = jax.lax.dot_general(x, t0_ref[...], (((2,), (1,)), ((1,), (0,))),
                             preferred_element_type=jnp.float32)
    return jnp.sum(pc, axis=0) + c0b_ref[0:1, 0:1]


def _prep_body(x_ref, t0_ref, c0b_ref, g1w_ref, pro_ref, s1_ref):
    pro = _conv0(x_ref[...], t0_ref, c0b_ref)
    pro_ref[...] = pro
    s1_ref[...] = _mm(pro, g1w_ref[...])


def _layer1_body(adj_ref, s_ref, g3w_ref, pro_ref, eps_ref,
                 f1w_ref, f1b_ref, f2w_ref, f2b_ref, f31w_ref, f31b_ref,
                 f21w_ref, f21b_ref, f22w_ref, f22b_ref,
                 s2_ref, z_ref, mu_ref, logvar_ref):
    h = jax.nn.relu(_mm(adj_ref[...], s_ref[...]))
    s2_ref[...] = _mm(h, g3w_ref[...])
    pro = pro_ref[...]
    h1 = jax.nn.relu(_mm(pro, f1w_ref[...]) + f1b_ref[...])
    h2 = jax.nn.relu(_mm(h1, f2w_ref[...]) + f2b_ref[...])
    h3 = jax.nn.relu(_mm(h2, f31w_ref[...]) + f31b_ref[...])
    mu = _mm(h3, f21w_ref[...]) + f21b_ref[...]
    logvar = _mm(h3, f22w_ref[...]) + f22b_ref[...]
    std = jnp.exp(0.5 * logvar)
    z_ref[...] = eps_ref[...] * std + mu
    mu_ref[...] = mu
    logvar_ref[...] = logvar


def _layer2_body(adj_ref, s_ref, g4w_ref, z_ref, t1_ref, c1b_ref,
                 f3w_ref, f3b_ref, f32w_ref, f32b_ref, f4w_ref, f4b_ref,
                 s3_ref, out0_ref):
    h = jax.nn.relu(_mm(adj_ref[...], s_ref[...]))
    s3_ref[...] = _mm(h, g4w_ref[...])
    d3 = jax.nn.relu(_mm(z_ref[...], f3w_ref[...]) + f3b_ref[...])
    d4 = jax.nn.relu(_mm(d3, f32w_ref[...]) + f32b_ref[...])
    recon = jax.nn.sigmoid(_mm(d4, f4w_ref[...]) + f4b_ref[...])
    for co in range(VAR):
        out0_ref[:, co, :] = _mm(recon, t1_ref[co]) + c1b_ref[0:1, co:co + 1]


def _gnn_body(adj_ref, s_ref, w_ref, b_ref, out_ref, *, act, last):
    h = _mm(adj_ref[...], s_ref[...])
    if act:
        h = jax.nn.relu(h)
    y = _mm(h, w_ref[...])
    if last:
        logits = y + b_ref[...]
        m = jnp.max(logits, axis=1, keepdims=True)
        e = jnp.exp(logits - m)
        out_ref[...] = e / jnp.sum(e, axis=1, keepdims=True)
    else:
        out_ref[...] = y


def _full_spec(shape):
    nd = len(shape)
    return pl.BlockSpec(shape, lambda i, _n=nd: (0,) * _n)


def _gnn_layer(adj, s, w, b, *, act, last):
    nb = N // _BM_G
    out_cols = NC if last else s.shape[1]
    body = functools.partial(_gnn_body, act=act, last=last)
    return pl.pallas_call(
        body,
        grid=(nb,),
        in_specs=[
            pl.BlockSpec((_BM_G, N), lambda i: (i, 0)),
            _full_spec(s.shape),
            _full_spec(w.shape),
            _full_spec(b.shape),
        ],
        out_specs=pl.BlockSpec((_BM_G, out_cols), lambda i: (i, 0)),
        out_shape=jax.ShapeDtypeStruct((N, out_cols), jnp.float32),
    )(adj, s, w, b)


def kernel(x, adj, eps, conv0_w, conv0_b, fc1_w, fc1_b, fc2_w, fc2_b,
           fc31_w, fc31_b, fc21_w, fc21_b, fc22_w, fc22_b, fc3_w, fc3_b,
           fc32_w, fc32_b, fc4_w, fc4_b, conv1_w, conv1_b,
           g1_w, g3_w, g4_w, g5_w, fcc_w, fcc_b):
    f32 = jnp.float32
    c0w = conv0_w.reshape(VAR, 3)
    c0b = conv0_b.reshape(1, 1)
    c1w = conv1_w.reshape(VAR, 3)
    c1b = conv1_b.reshape(1, VAR)
    e_up = jnp.eye(NIN, k=1, dtype=f32)
    e_d = jnp.eye(NIN, dtype=f32)
    e_dn = jnp.eye(NIN, k=-1, dtype=f32)
    t0 = (c0w[:, 0, None, None] * e_up + c0w[:, 1, None, None] * e_d
          + c0w[:, 2, None, None] * e_dn)
    t1 = (c1w[:, 0, None, None] * e_up + c1w[:, 1, None, None] * e_d
          + c1w[:, 2, None, None] * e_dn)

    nbp = N // _BM_PREP
    pro_x, s1 = pl.pallas_call(
        _prep_body,
        grid=(nbp,),
        in_specs=[
            pl.BlockSpec((_BM_PREP, VAR, NIN), lambda i: (i, 0, 0)),
            _full_spec(t0.shape),
            _full_spec(c0b.shape),
            _full_spec(g1_w.shape),
        ],
        out_specs=[
            pl.BlockSpec((_BM_PREP, NIN), lambda i: (i, 0)),
            pl.BlockSpec((_BM_PREP, NZ), lambda i: (i, 0)),
        ],
        out_shape=[
            jax.ShapeDtypeStruct((N, NIN), f32),
            jax.ShapeDtypeStruct((N, NZ), f32),
        ],
    )(x, t0, c0b, g1_w)

    nb = N // _BM_G
    biases = dict(
        f1b=fc1_b.reshape(1, -1), f2b=fc2_b.reshape(1, -1),
        f31b=fc31_b.reshape(1, -1), f21b=fc21_b.reshape(1, -1),
        f22b=fc22_b.reshape(1, -1), f3b=fc3_b.reshape(1, -1),
        f32b=fc32_b.reshape(1, -1), f4b=fc4_b.reshape(1, -1),
    )
    l1_inputs = (adj, s1, g3_w, pro_x, eps,
                 fc1_w, biases['f1b'], fc2_w, biases['f2b'],
                 fc31_w, biases['f31b'], fc21_w, biases['f21b'],
                 fc22_w, biases['f22b'])
    in_specs = [
        pl.BlockSpec((_BM_G, N), lambda i: (i, 0)),
        _full_spec(s1.shape),
        _full_spec(g3_w.shape),
        pl.BlockSpec((_BM_G, NIN), lambda i: (i, 0)),
        pl.BlockSpec((_BM_G, NZ), lambda i: (i, 0)),
    ] + [_full_spec(a.shape) for a in l1_inputs[5:]]
    out_specs = [
        pl.BlockSpec((_BM_G, NZ), lambda i: (i, 0)),
        pl.BlockSpec((_BM_G, NZ), lambda i: (i, 0)),
        pl.BlockSpec((_BM_G, NZ), lambda i: (i, 0)),
        pl.BlockSpec((_BM_G, NZ), lambda i: (i, 0)),
    ]
    out_shape = [
        jax.ShapeDtypeStruct((N, NZ), f32),
        jax.ShapeDtypeStruct((N, NZ), f32),
        jax.ShapeDtypeStruct((N, NZ), f32),
        jax.ShapeDtypeStruct((N, NZ), f32),
    ]
    s2, z, mu, logvar = pl.pallas_call(
        _layer1_body,
        grid=(nb,),
        in_specs=in_specs,
        out_specs=out_specs,
        out_shape=out_shape,
    )(*l1_inputs)

    l2_inputs = (adj, s2, g4_w, z, t1, c1b,
                 fc3_w, biases['f3b'], fc32_w, biases['f32b'],
                 fc4_w, biases['f4b'])
    in_specs2 = [
        pl.BlockSpec((_BM_G, N), lambda i: (i, 0)),
        _full_spec(s2.shape),
        _full_spec(g4_w.shape),
        pl.BlockSpec((_BM_G, NZ), lambda i: (i, 0)),
    ] + [_full_spec(a.shape) for a in l2_inputs[4:]]
    out_specs2 = [
        pl.BlockSpec((_BM_G, NZ), lambda i: (i, 0)),
        pl.BlockSpec((_BM_G, VAR, NIN), lambda i: (i, 0, 0)),
    ]
    out_shape2 = [
        jax.ShapeDtypeStruct((N, NZ), f32),
        jax.ShapeDtypeStruct((N, VAR, NIN), f32),
    ]
    s3, out0 = pl.pallas_call(
        _layer2_body,
        grid=(nb,),
        in_specs=in_specs2,
        out_specs=out_specs2,
        out_shape=out_shape2,
    )(*l2_inputs)

    dummy_b = jnp.zeros((1, 1), f32)
    s4 = _gnn_layer(adj, s3, g5_w, dummy_b, act=False, last=False)
    predict = _gnn_layer(adj, s4, fcc_w, fcc_b.reshape(1, NC), act=False,
                         last=True)

    return (out0, predict, mu, logvar)


# speedup vs baseline: 1.0703x; 1.0703x over previous
import functools

import jax
import jax.numpy as jnp
from jax.experimental import pallas as pl

N = 10000
VAR = 4
NIN = 256
NZ = 100
NC = 10

_BM_PREP = 1000
_BM_G = 400


def _mm(a, b):
    return jax.lax.dot_general(a, b, (((1,), (0,)), ((), ())),
                               preferred_element_type=jnp.float32)


def _conv0(x, t0_ref, c0b_ref):
    pc = jax.lax.dot_general(x, t0_ref[...], (((2,), (1,)), ((1,), (0,))),
                             preferred_element_type=jnp.float32)
    return jnp.sum(pc, axis=0) + c0b_ref[0:1, 0:1]


def _prep_body(x_ref, t0_ref, c0b_ref, g1w_ref, pro_ref, s1_ref):
    pro = _conv0(x_ref[...], t0_ref, c0b_ref)
    pro_ref[...] = pro
    s1_ref[...] = _mm(pro, g1w_ref[...])


def _layer1_body(adj_ref, s_ref, g3w_ref, pro_ref, eps_ref,
                 f1w_ref, f1b_ref, f2w_ref, f2b_ref, f31w_ref, f31b_ref,
                 f21w_ref, f21b_ref, f22w_ref, f22b_ref,
                 s2_ref, z_ref, mu_ref, logvar_ref):
    h = jax.nn.relu(_mm(adj_ref[...], s_ref[...]))
    s2_ref[...] = _mm(h, g3w_ref[...])
    pro = pro_ref[...]
    h1 = jax.nn.relu(_mm(pro, f1w_ref[...]) + f1b_ref[...])
    h2 = jax.nn.relu(_mm(h1, f2w_ref[...]) + f2b_ref[...])
    h3 = jax.nn.relu(_mm(h2, f31w_ref[...]) + f31b_ref[...])
    mu = _mm(h3, f21w_ref[...]) + f21b_ref[...]
    logvar = _mm(h3, f22w_ref[...]) + f22b_ref[...]
    std = jnp.exp(0.5 * logvar)
    z_ref[...] = eps_ref[...] * std + mu
    mu_ref[...] = mu
    logvar_ref[...] = logvar


def _layer2_body(adj_ref, s_ref, g4w_ref, z_ref, t1_ref, c1b_ref,
                 f3w_ref, f3b_ref, f32w_ref, f32b_ref, f4w_ref, f4b_ref,
                 s3_ref, out0_ref):
    h = jax.nn.relu(_mm(adj_ref[...], s_ref[...]))
    s3_ref[...] = _mm(h, g4w_ref[...])
    d3 = jax.nn.relu(_mm(z_ref[...], f3w_ref[...]) + f3b_ref[...])
    d4 = jax.nn.relu(_mm(d3, f32w_ref[...]) + f32b_ref[...])
    recon = jax.nn.sigmoid(_mm(d4, f4w_ref[...]) + f4b_ref[...])
    for co in range(VAR):
        out0_ref[:, co, :] = _mm(recon, t1_ref[co]) + c1b_ref[0:1, co:co + 1]


def _gnn_body(adj_ref, s_ref, w_ref, b_ref, out_ref, *, act, last):
    h = _mm(adj_ref[...], s_ref[...])
    if act:
        h = jax.nn.relu(h)
    y = _mm(h, w_ref[...])
    if last:
        logits = y + b_ref[...]
        m = jnp.max(logits, axis=1, keepdims=True)
        e = jnp.exp(logits - m)
        out_ref[...] = e / jnp.sum(e, axis=1, keepdims=True)
    else:
        out_ref[...] = y


def _full_spec(shape):
    nd = len(shape)
    return pl.BlockSpec(shape, lambda i, _n=nd: (0,) * _n)


def _gnn_layer(adj, s, w, b, *, act, last, bm=_BM_G):
    nb = -(-N // bm)
    out_cols = NC if last else s.shape[1]
    body = functools.partial(_gnn_body, act=act, last=last)
    return pl.pallas_call(
        body,
        grid=(nb,),
        in_specs=[
            pl.BlockSpec((bm, N), lambda i: (i, 0)),
            _full_spec(s.shape),
            _full_spec(w.shape),
            _full_spec(b.shape),
        ],
        out_specs=pl.BlockSpec((bm, out_cols), lambda i: (i, 0)),
        out_shape=jax.ShapeDtypeStruct((N, out_cols), jnp.float32),
    )(adj, s, w, b)


def kernel(x, adj, eps, conv0_w, conv0_b, fc1_w, fc1_b, fc2_w, fc2_b,
           fc31_w, fc31_b, fc21_w, fc21_b, fc22_w, fc22_b, fc3_w, fc3_b,
           fc32_w, fc32_b, fc4_w, fc4_b, conv1_w, conv1_b,
           g1_w, g3_w, g4_w, g5_w, fcc_w, fcc_b):
    f32 = jnp.float32
    c0w = conv0_w.reshape(VAR, 3)
    c0b = conv0_b.reshape(1, 1)
    c1w = conv1_w.reshape(VAR, 3)
    c1b = conv1_b.reshape(1, VAR)
    e_up = jnp.eye(NIN, k=1, dtype=f32)
    e_d = jnp.eye(NIN, dtype=f32)
    e_dn = jnp.eye(NIN, k=-1, dtype=f32)
    t0 = (c0w[:, 0, None, None] * e_up + c0w[:, 1, None, None] * e_d
          + c0w[:, 2, None, None] * e_dn)
    t1 = (c1w[:, 0, None, None] * e_up + c1w[:, 1, None, None] * e_d
          + c1w[:, 2, None, None] * e_dn)

    nbp = N // _BM_PREP
    pro_x, s1 = pl.pallas_call(
        _prep_body,
        grid=(nbp,),
        in_specs=[
            pl.BlockSpec((_BM_PREP, VAR, NIN), lambda i: (i, 0, 0)),
            _full_spec(t0.shape),
            _full_spec(c0b.shape),
            _full_spec(g1_w.shape),
        ],
        out_specs=[
            pl.BlockSpec((_BM_PREP, NIN), lambda i: (i, 0)),
            pl.BlockSpec((_BM_PREP, NZ), lambda i: (i, 0)),
        ],
        out_shape=[
            jax.ShapeDtypeStruct((N, NIN), f32),
            jax.ShapeDtypeStruct((N, NZ), f32),
        ],
    )(x, t0, c0b, g1_w)

    nb = N // _BM_G
    biases = dict(
        f1b=fc1_b.reshape(1, -1), f2b=fc2_b.reshape(1, -1),
        f31b=fc31_b.reshape(1, -1), f21b=fc21_b.reshape(1, -1),
        f22b=fc22_b.reshape(1, -1), f3b=fc3_b.reshape(1, -1),
        f32b=fc32_b.reshape(1, -1), f4b=fc4_b.reshape(1, -1),
    )
    l1_inputs = (adj, s1, g3_w, pro_x, eps,
                 fc1_w, biases['f1b'], fc2_w, biases['f2b'],
                 fc31_w, biases['f31b'], fc21_w, biases['f21b'],
                 fc22_w, biases['f22b'])
    in_specs = [
        pl.BlockSpec((_BM_G, N), lambda i: (i, 0)),
        _full_spec(s1.shape),
        _full_spec(g3_w.shape),
        pl.BlockSpec((_BM_G, NIN), lambda i: (i, 0)),
        pl.BlockSpec((_BM_G, NZ), lambda i: (i, 0)),
    ] + [_full_spec(a.shape) for a in l1_inputs[5:]]
    out_specs = [
        pl.BlockSpec((_BM_G, NZ), lambda i: (i, 0)),
        pl.BlockSpec((_BM_G, NZ), lambda i: (i, 0)),
        pl.BlockSpec((_BM_G, NZ), lambda i: (i, 0)),
        pl.BlockSpec((_BM_G, NZ), lambda i: (i, 0)),
    ]
    out_shape = [
        jax.ShapeDtypeStruct((N, NZ), f32),
        jax.ShapeDtypeStruct((N, NZ), f32),
        jax.ShapeDtypeStruct((N, NZ), f32),
        jax.ShapeDtypeStruct((N, NZ), f32),
    ]
    s2, z, mu, logvar = pl.pallas_call(
        _layer1_body,
        grid=(nb,),
        in_specs=in_specs,
        out_specs=out_specs,
        out_shape=out_shape,
    )(*l1_inputs)

    l2_inputs = (adj, s2, g4_w, z, t1, c1b,
                 fc3_w, biases['f3b'], fc32_w, biases['f32b'],
                 fc4_w, biases['f4b'])
    in_specs2 = [
        pl.BlockSpec((_BM_G, N), lambda i: (i, 0)),
        _full_spec(s2.shape),
        _full_spec(g4_w.shape),
        pl.BlockSpec((_BM_G, NZ), lambda i: (i, 0)),
    ] + [_full_spec(a.shape) for a in l2_inputs[4:]]
    out_specs2 = [
        pl.BlockSpec((_BM_G, NZ), lambda i: (i, 0)),
        pl.BlockSpec((_BM_G, VAR, NIN), lambda i: (i, 0, 0)),
    ]
    out_shape2 = [
        jax.ShapeDtypeStruct((N, NZ), f32),
        jax.ShapeDtypeStruct((N, VAR, NIN), f32),
    ]
    s3, out0 = pl.pallas_call(
        _layer2_body,
        grid=(nb,),
        in_specs=in_specs2,
        out_specs=out_specs2,
        out_shape=out_shape2,
    )(*l2_inputs)

    dummy_b = jnp.zeros((1, 1), f32)
    s4 = _gnn_layer(adj, s3, g5_w, dummy_b, act=False, last=False, bm=592)
    predict = _gnn_layer(adj, s4, fcc_w, fcc_b.reshape(1, NC), act=False,
                         last=True, bm=592)

    return (out0, predict, mu, logvar)


# speedup vs baseline: 1.0897x; 1.0181x over previous
import jax
import jax.numpy as jnp
from jax.experimental import pallas as pl
from jax.experimental.pallas import tpu as pltpu

N = 10000
VAR = 4
NIN = 256
NZ = 100
NC = 10

_BM_AE = 1000
_BM_G = 400
_NBG = N // _BM_G


def _mm(a, b):
    return jax.lax.dot_general(a, b, (((1,), (0,)), ((), ())),
                               preferred_element_type=jnp.float32)


def _ae_body(x_ref, eps_ref, t0_ref, c0b_ref, t1_ref, c1b_ref,
             f1w_ref, f1b_ref, f2w_ref, f2b_ref, f31w_ref, f31b_ref,
             f21w_ref, f21b_ref, f22w_ref, f22b_ref,
             f3w_ref, f3b_ref, f32w_ref, f32b_ref, f4w_ref, f4b_ref,
             g1w_ref,
             out0_ref, mu_ref, logvar_ref, s1_ref):
    x = x_ref[...]
    pc = jax.lax.dot_general(x, t0_ref[...], (((2,), (1,)), ((1,), (0,))),
                             preferred_element_type=jnp.float32)
    pro = jnp.sum(pc, axis=0) + c0b_ref[0:1, 0:1]
    h1 = jax.nn.relu(_mm(pro, f1w_ref[...]) + f1b_ref[...])
    h2 = jax.nn.relu(_mm(h1, f2w_ref[...]) + f2b_ref[...])
    h3 = jax.nn.relu(_mm(h2, f31w_ref[...]) + f31b_ref[...])
    mu = _mm(h3, f21w_ref[...]) + f21b_ref[...]
    logvar = _mm(h3, f22w_ref[...]) + f22b_ref[...]
    std = jnp.exp(0.5 * logvar)
    z = eps_ref[...] * std + mu
    d3 = jax.nn.relu(_mm(z, f3w_ref[...]) + f3b_ref[...])
    d4 = jax.nn.relu(_mm(d3, f32w_ref[...]) + f32b_ref[...])
    recon = jax.nn.sigmoid(_mm(d4, f4w_ref[...]) + f4b_ref[...])
    for co in range(VAR):
        out0_ref[:, co, :] = _mm(recon, t1_ref[co]) + c1b_ref[0:1, co:co + 1]
    mu_ref[...] = mu
    logvar_ref[...] = logvar
    s1_ref[...] = _mm(pro, g1w_ref[...])


def _gnn_body(adj_ref, s1_ref, g3w_ref, g4w_ref, g5w_ref, fccw_ref, fccb_ref,
              pred_ref, sa_ref, sb_ref):
    l = pl.program_id(0)
    i = pl.program_id(1)
    rows = pl.ds(i * _BM_G, _BM_G)

    @pl.when(l == 0)
    def _():
        sa_ref[rows, :] = s1_ref[...]

    @pl.when(l == 1)
    def _():
        h = jax.nn.relu(_mm(adj_ref[...], sa_ref[...]))
        sb_ref[rows, :] = _mm(h, g3w_ref[...])

    @pl.when(l == 2)
    def _():
        h = jax.nn.relu(_mm(adj_ref[...], sb_ref[...]))
        sa_ref[rows, :] = _mm(h, g4w_ref[...])

    @pl.when(l == 3)
    def _():
        h = _mm(adj_ref[...], sa_ref[...])
        sb_ref[rows, :] = _mm(h, g5w_ref[...])

    @pl.when(l == 4)
    def _():
        h = _mm(adj_ref[...], sb_ref[...])
        logits = _mm(h, fccw_ref[...]) + fccb_ref[...]
        m = jnp.max(logits, axis=1, keepdims=True)
        e = jnp.exp(logits - m)
        pred_ref[...] = e / jnp.sum(e, axis=1, keepdims=True)


def _full_spec(shape):
    nd = len(shape)
    return pl.BlockSpec(shape, lambda l, i, _n=nd: (0,) * _n)


def kernel(x, adj, eps, conv0_w, conv0_b, fc1_w, fc1_b, fc2_w, fc2_b,
           fc31_w, fc31_b, fc21_w, fc21_b, fc22_w, fc22_b, fc3_w, fc3_b,
           fc32_w, fc32_b, fc4_w, fc4_b, conv1_w, conv1_b,
           g1_w, g3_w, g4_w, g5_w, fcc_w, fcc_b):
    f32 = jnp.float32
    c0w = conv0_w.reshape(VAR, 3)
    c0b = conv0_b.reshape(1, 1)
    c1w = conv1_w.reshape(VAR, 3)
    c1b = conv1_b.reshape(1, VAR)
    e_up = jnp.eye(NIN, k=1, dtype=f32)
    e_d = jnp.eye(NIN, dtype=f32)
    e_dn = jnp.eye(NIN, k=-1, dtype=f32)
    t0 = (c0w[:, 0, None, None] * e_up + c0w[:, 1, None, None] * e_d
          + c0w[:, 2, None, None] * e_dn)
    t1 = (c1w[:, 0, None, None] * e_up + c1w[:, 1, None, None] * e_d
          + c1w[:, 2, None, None] * e_dn)

    nb = N // _BM_AE
    ae_inputs = (x, eps, t0, c0b, t1, c1b,
                 fc1_w, fc1_b.reshape(1, -1), fc2_w, fc2_b.reshape(1, -1),
                 fc31_w, fc31_b.reshape(1, -1), fc21_w, fc21_b.reshape(1, -1),
                 fc22_w, fc22_b.reshape(1, -1), fc3_w, fc3_b.reshape(1, -1),
                 fc32_w, fc32_b.reshape(1, -1), fc4_w, fc4_b.reshape(1, -1),
                 g1_w)
    ae_in_specs = [
        pl.BlockSpec((_BM_AE, VAR, NIN), lambda i: (i, 0, 0)),
        pl.BlockSpec((_BM_AE, NZ), lambda i: (i, 0)),
    ] + [pl.BlockSpec(a.shape, lambda i, _n=a.ndim: (0,) * _n)
         for a in ae_inputs[2:]]
    out0, mu, logvar, s1 = pl.pallas_call(
        _ae_body,
        grid=(nb,),
        in_specs=ae_in_specs,
        out_specs=[
            pl.BlockSpec((_BM_AE, VAR, NIN), lambda i: (i, 0, 0)),
            pl.BlockSpec((_BM_AE, NZ), lambda i: (i, 0)),
            pl.BlockSpec((_BM_AE, NZ), lambda i: (i, 0)),
            pl.BlockSpec((_BM_AE, NZ), lambda i: (i, 0)),
        ],
        out_shape=[
            jax.ShapeDtypeStruct((N, VAR, NIN), f32),
            jax.ShapeDtypeStruct((N, NZ), f32),
            jax.ShapeDtypeStruct((N, NZ), f32),
            jax.ShapeDtypeStruct((N, NZ), f32),
        ],
    )(*ae_inputs)

    gnn_inputs = (adj, s1, g3_w, g4_w, g5_w, fcc_w, fcc_b.reshape(1, NC))
    predict = pl.pallas_call(
        _gnn_body,
        grid=(5, _NBG),
        in_specs=[
            pl.BlockSpec((_BM_G, N),
                         lambda l, i: (jax.lax.select(l == 0, 0, i), 0)),
            pl.BlockSpec((_BM_G, NZ),
                         lambda l, i: (jax.lax.select(l == 0, i, 0), 0)),
        ] + [_full_spec(a.shape) for a in gnn_inputs[2:]],
        out_specs=pl.BlockSpec((_BM_G, NC), lambda l, i: (i, 0)),
        out_shape=jax.ShapeDtypeStruct((N, NC), f32),
        scratch_shapes=[
            pltpu.VMEM((N, NZ), f32),
            pltpu.VMEM((N, NZ), f32),
        ],
    )(*gnn_inputs)

    return (out0, predict, mu, logvar)


# speedup vs baseline: 1.0904x; 1.0006x over previous
import jax
import jax.numpy as jnp
from jax.experimental import pallas as pl
from jax.experimental.pallas import tpu as pltpu

N = 10000
VAR = 4
NIN = 256
NZ = 100
NC = 10

_BM_AE = 1000
_BM_G = 400
_NBG = N // _BM_G


def _mm(a, b):
    return jax.lax.dot_general(a, b, (((1,), (0,)), ((), ())),
                               preferred_element_type=jnp.float32)


def _ae_body(x_ref, eps_ref, t0_ref, c0b_ref, t1_ref, c1b_ref,
             f1w_ref, f1b_ref, f2w_ref, f2b_ref, f31w_ref, f31b_ref,
             f21w_ref, f21b_ref, f22w_ref, f22b_ref,
             f3w_ref, f3b_ref, f32w_ref, f32b_ref, f4w_ref, f4b_ref,
             g1w_ref,
             out0_ref, mu_ref, logvar_ref, s1_ref):
    x = x_ref[...]
    pc = jax.lax.dot_general(x, t0_ref[...], (((2,), (1,)), ((1,), (0,))),
                             preferred_element_type=jnp.float32)
    pro = jnp.sum(pc, axis=0) + c0b_ref[0:1, 0:1]
    h1 = jax.nn.relu(_mm(pro, f1w_ref[...]) + f1b_ref[...])
    h2 = jax.nn.relu(_mm(h1, f2w_ref[...]) + f2b_ref[...])
    h3 = jax.nn.relu(_mm(h2, f31w_ref[...]) + f31b_ref[...])
    mu = _mm(h3, f21w_ref[...]) + f21b_ref[...]
    logvar = _mm(h3, f22w_ref[...]) + f22b_ref[...]
    std = jnp.exp(0.5 * logvar)
    z = eps_ref[...] * std + mu
    d3 = jax.nn.relu(_mm(z, f3w_ref[...]) + f3b_ref[...])
    d4 = jax.nn.relu(_mm(d3, f32w_ref[...]) + f32b_ref[...])
    recon = jax.nn.sigmoid(_mm(d4, f4w_ref[...]) + f4b_ref[...])
    for co in range(VAR):
        out0_ref[:, co, :] = _mm(recon, t1_ref[co]) + c1b_ref[0:1, co:co + 1]
    mu_ref[...] = mu
    logvar_ref[...] = logvar
    s1_ref[...] = _mm(pro, g1w_ref[...])


def _gnn_body(adja_ref, adjb_ref, s1_ref, g3w_ref, g4w_ref, g5w_ref,
              fccw_ref, fccb_ref, pred_ref, sa_ref, sb_ref):
    l = pl.program_id(0)
    i = pl.program_id(1)
    rows = pl.ds(i * _BM_G, _BM_G)

    def adj_mm(s_ref):
        return jnp.concatenate(
            [_mm(adja_ref[...], s_ref[...]), _mm(adjb_ref[...], s_ref[...])],
            axis=0)

    @pl.when(l == 0)
    def _():
        sa_ref[rows, :] = s1_ref[...]

    @pl.when(l == 1)
    def _():
        h = jax.nn.relu(adj_mm(sa_ref))
        sb_ref[rows, :] = _mm(h, g3w_ref[...])

    @pl.when(l == 2)
    def _():
        h = jax.nn.relu(adj_mm(sb_ref))
        sa_ref[rows, :] = _mm(h, g4w_ref[...])

    @pl.when(l == 3)
    def _():
        h = adj_mm(sa_ref)
        sb_ref[rows, :] = _mm(h, g5w_ref[...])

    @pl.when(l == 4)
    def _():
        h = adj_mm(sb_ref)
        logits = _mm(h, fccw_ref[...]) + fccb_ref[...]
        m = jnp.max(logits, axis=1, keepdims=True)
        e = jnp.exp(logits - m)
        pred_ref[...] = e / jnp.sum(e, axis=1, keepdims=True)


def _full_spec(shape):
    nd = len(shape)
    return pl.BlockSpec(shape, lambda l, i, _n=nd: (0,) * _n)


def kernel(x, adj, eps, conv0_w, conv0_b, fc1_w, fc1_b, fc2_w, fc2_b,
           fc31_w, fc31_b, fc21_w, fc21_b, fc22_w, fc22_b, fc3_w, fc3_b,
           fc32_w, fc32_b, fc4_w, fc4_b, conv1_w, conv1_b,
           g1_w, g3_w, g4_w, g5_w, fcc_w, fcc_b):
    f32 = jnp.float32
    c0w = conv0_w.reshape(VAR, 3)
    c0b = conv0_b.reshape(1, 1)
    c1w = conv1_w.reshape(VAR, 3)
    c1b = conv1_b.reshape(1, VAR)
    e_up = jnp.eye(NIN, k=1, dtype=f32)
    e_d = jnp.eye(NIN, dtype=f32)
    e_dn = jnp.eye(NIN, k=-1, dtype=f32)
    t0 = (c0w[:, 0, None, None] * e_up + c0w[:, 1, None, None] * e_d
          + c0w[:, 2, None, None] * e_dn)
    t1 = (c1w[:, 0, None, None] * e_up + c1w[:, 1, None, None] * e_d
          + c1w[:, 2, None, None] * e_dn)

    nb = N // _BM_AE
    ae_inputs = (x, eps, t0, c0b, t1, c1b,
                 fc1_w, fc1_b.reshape(1, -1), fc2_w, fc2_b.reshape(1, -1),
                 fc31_w, fc31_b.reshape(1, -1), fc21_w, fc21_b.reshape(1, -1),
                 fc22_w, fc22_b.reshape(1, -1), fc3_w, fc3_b.reshape(1, -1),
                 fc32_w, fc32_b.reshape(1, -1), fc4_w, fc4_b.reshape(1, -1),
                 g1_w)
    ae_in_specs = [
        pl.BlockSpec((_BM_AE, VAR, NIN), lambda i: (i, 0, 0)),
        pl.BlockSpec((_BM_AE, NZ), lambda i: (i, 0)),
    ] + [pl.BlockSpec(a.shape, lambda i, _n=a.ndim: (0,) * _n)
         for a in ae_inputs[2:]]
    out0, mu, logvar, s1 = pl.pallas_call(
        _ae_body,
        grid=(nb,),
        in_specs=ae_in_specs,
        out_specs=[
            pl.BlockSpec((_BM_AE, VAR, NIN), lambda i: (i, 0, 0)),
            pl.BlockSpec((_BM_AE, NZ), lambda i: (i, 0)),
            pl.BlockSpec((_BM_AE, NZ), lambda i: (i, 0)),
            pl.BlockSpec((_BM_AE, NZ), lambda i: (i, 0)),
        ],
        out_shape=[
            jax.ShapeDtypeStruct((N, VAR, NIN), f32),
            jax.ShapeDtypeStruct((N, NZ), f32),
            jax.ShapeDtypeStruct((N, NZ), f32),
            jax.ShapeDtypeStruct((N, NZ), f32),
        ],
    )(*ae_inputs)

    gnn_inputs = (adj, adj, s1, g3_w, g4_w, g5_w, fcc_w, fcc_b.reshape(1, NC))
    predict = pl.pallas_call(
        _gnn_body,
        grid=(5, _NBG),
        in_specs=[
            pl.BlockSpec((_BM_G // 2, N),
                         lambda l, i: (jax.lax.select(l == 0, 0, 2 * i), 0)),
            pl.BlockSpec((_BM_G // 2, N),
                         lambda l, i: (jax.lax.select(l == 0, 1, 2 * i + 1), 0)),
            pl.BlockSpec((_BM_G, NZ),
                         lambda l, i: (jax.lax.select(l == 0, i, 0), 0)),
        ] + [_full_spec(a.shape) for a in gnn_inputs[3:]],
        out_specs=pl.BlockSpec((_BM_G, NC), lambda l, i: (i, 0)),
        out_shape=jax.ShapeDtypeStruct((N, NC), f32),
        scratch_shapes=[
            pltpu.VMEM((N, NZ), f32),
            pltpu.VMEM((N, NZ), f32),
        ],
    )(*gnn_inputs)

    return (out0, predict, mu, logvar)
